# Initial kernel scaffold; baseline (speedup 1.0000x reference)
import functools

import jax
import jax.numpy as jnp
from jax import lax
from jax.experimental import pallas as pl
from jax.experimental.pallas import tpu as pltpu
from jax.experimental.pallas import tpu_sc as plsc

N = 10000
E = 320000
IN_CH = 128
HID_CH = 128
OUT_CH = 64

NC = 2
NS = 16
NW = NC * NS
EW = E // NW
K = 80
NB = EW // K
RPT = N // NS
LANES = 16


def _mm_kernel(x_ref, w_ref, o_ref):
    o_ref[...] = jnp.dot(x_ref[...], w_ref[...],
                         preferred_element_type=jnp.float32,
                         precision=lax.Precision.HIGHEST)


def _tc_matmul(x, w, block_rows=2000):
    n, d_in = x.shape
    d_out = w.shape[1]
    return pl.pallas_call(
        _mm_kernel,
        grid=(n // block_rows,),
        in_specs=[
            pl.BlockSpec((block_rows, d_in), lambda i: (i, 0)),
            pl.BlockSpec((d_in, d_out), lambda i: (0, 0)),
        ],
        out_specs=pl.BlockSpec((block_rows, d_out), lambda i: (i, 0)),
        out_shape=jax.ShapeDtypeStruct((n, d_out), jnp.float32),
    )(x, w)


def _bcast_kernel(a_ref, o_ref):
    o_ref[...] = jnp.broadcast_to(a_ref[...][:, None], o_ref.shape)


def _tc_attr_bcast(attr, block=8000):
    return pl.pallas_call(
        _bcast_kernel,
        grid=(E // block,),
        in_specs=[pl.BlockSpec((block,), lambda i: (i,))],
        out_specs=pl.BlockSpec((block, LANES), lambda i: (i, 0)),
        out_shape=jax.ShapeDtypeStruct((E, LANES), jnp.float32),
    )(attr)


def _combine_mm_kernel(p_ref, b_ref, w_ref, o_ref):
    h = jnp.maximum(p_ref[0] + p_ref[1] + b_ref[...], 0.0)
    o_ref[...] = jnp.dot(h, w_ref[...],
                         preferred_element_type=jnp.float32,
                         precision=lax.Precision.HIGHEST)


def _tc_combine_matmul(partials, b, w, block_rows=2000):
    _, n, d_in = partials.shape
    d_out = w.shape[1]
    return pl.pallas_call(
        _combine_mm_kernel,
        grid=(n // block_rows,),
        in_specs=[
            pl.BlockSpec((2, block_rows, d_in), lambda i: (0, i, 0)),
            pl.BlockSpec((1, d_in), lambda i: (0, 0)),
            pl.BlockSpec((d_in, d_out), lambda i: (0, 0)),
        ],
        out_specs=pl.BlockSpec((block_rows, d_out), lambda i: (i, 0)),
        out_shape=jax.ShapeDtypeStruct((n, d_out), jnp.float32),
    )(partials, b.reshape(1, d_in), w)


def _finish_kernel(p_ref, b_ref, o_ref):
    o_ref[...] = p_ref[0] + p_ref[1] + b_ref[...]


def _tc_finish(partials, b, block_rows=2000):
    _, n, d = partials.shape
    return pl.pallas_call(
        _finish_kernel,
        grid=(n // block_rows,),
        in_specs=[
            pl.BlockSpec((2, block_rows, d), lambda i: (0, i, 0)),
            pl.BlockSpec((1, d), lambda i: (0, 0)),
        ],
        out_specs=pl.BlockSpec((block_rows, d), lambda i: (i, 0)),
        out_shape=jax.ShapeDtypeStruct((n, d), jnp.float32),
    )(partials, b.reshape(1, d))


def _make_propagate(d):
    mesh = plsc.VectorSubcoreMesh(core_axis_name="c", subcore_axis_name="s")
    nch = d // LANES

    @functools.partial(
        pl.kernel,
        out_type=jax.ShapeDtypeStruct((NC, N, d), jnp.float32),
        mesh=mesh,
        scratch_types=[
            pltpu.VMEM((K,), jnp.int32),
            pltpu.VMEM((K,), jnp.int32),
            pltpu.VMEM((K, LANES), jnp.float32),
            pltpu.VMEM((K, d), jnp.float32),
            pltpu.VMEM_SHARED((N, d), jnp.float32),
            pltpu.SemaphoreType.DMA,
        ],
    )
    def prop(sup_hbm, src_hbm, dst_hbm, attr_hbm, out_hbm,
             srcv, dstv, attrv, rows, acc, sem):
        c = lax.axis_index("c")
        s = lax.axis_index("s")
        w = s * NC + c

        zero = jnp.zeros((LANES,), jnp.float32)

        @pl.loop(0, K)
        def _(r):
            for ch in range(nch):
                rows[r, pl.ds(ch * LANES, LANES)] = zero

        base_row = s * RPT
        nfull, rem = divmod(RPT, K)
        for i in range(nfull):
            pltpu.sync_copy(rows, acc.at[pl.ds(base_row + i * K, K)])
        if rem:
            pltpu.sync_copy(rows.at[pl.ds(0, rem)],
                            acc.at[pl.ds(base_row + nfull * K, rem)])
        plsc.subcore_barrier()

        ebase = w * EW

        @pl.loop(0, NB)
        def _(bi):
            off = ebase + bi * K
            pltpu.sync_copy(src_hbm.at[pl.ds(off, K)], srcv)
            pltpu.sync_copy(dst_hbm.at[pl.ds(off, K)], dstv)
            pltpu.sync_copy(attr_hbm.at[pl.ds(off, K)], attrv)
            pltpu.async_copy(sup_hbm.at[srcv], rows, sem).wait()

            @pl.loop(0, K)
            def _(e):
                a = attrv[e, pl.ds(0, LANES)]
                for ch in range(nch):
                    sl = (e, pl.ds(ch * LANES, LANES))
                    rows[sl] = rows[sl] * a

            pltpu.sync_copy(rows, acc.at[dstv], add=True)

        plsc.subcore_barrier()

        for i in range(nfull):
            pltpu.sync_copy(acc.at[pl.ds(base_row + i * K, K)],
                            out_hbm.at[c, pl.ds(base_row + i * K, K)])
        if rem:
            pltpu.sync_copy(acc.at[pl.ds(base_row + nfull * K, rem)],
                            out_hbm.at[c, pl.ds(base_row + nfull * K, rem)])

    return prop


_propagate_hid = _make_propagate(HID_CH)
_propagate_out = _make_propagate(OUT_CH)


def kernel(x, edge_index, edge_attr, W1, b1, W2, b2):
    src = edge_index[0].astype(jnp.int32)
    dst = edge_index[1].astype(jnp.int32)
    attr16 = _tc_attr_bcast(edge_attr)

    support1 = _tc_matmul(x, W1)
    partials1 = _propagate_hid(support1, src, dst, attr16)
    support2 = _tc_combine_matmul(partials1, b1, W2)
    partials2 = _propagate_out(support2, src, dst, attr16)
    return _tc_finish(partials2, b2)


# speedup vs baseline: 3.0680x; 3.0680x over previous
import functools

import jax
import jax.numpy as jnp
from jax import lax
from jax.experimental import pallas as pl
from jax.experimental.pallas import tpu as pltpu
from jax.experimental.pallas import tpu_sc as plsc

N = 10000
E = 320000
IN_CH = 128
HID_CH = 128
OUT_CH = 64

NC = 2
NS = 16
NW = NC * NS
EW = E // NW
K = 80
NB = EW // K
NP = 10240
RPT = NP // NS
LANES = 16
D = HID_CH


def _mm_kernel(x_ref, w_ref, o_ref):
    o_ref[...] = jnp.dot(x_ref[...], w_ref[...],
                         preferred_element_type=jnp.float32,
                         precision=lax.Precision.HIGHEST)


def _tc_matmul(x, w, block_rows=2000):
    n, d_in = x.shape
    d_out = w.shape[1]
    return pl.pallas_call(
        _mm_kernel,
        grid=(n // block_rows,),
        in_specs=[
            pl.BlockSpec((block_rows, d_in), lambda i: (i, 0)),
            pl.BlockSpec((d_in, d_out), lambda i: (0, 0)),
        ],
        out_specs=pl.BlockSpec((block_rows, d_out), lambda i: (i, 0)),
        out_shape=jax.ShapeDtypeStruct((n, d_out), jnp.float32),
    )(x, w)


def _relu_combine_kernel(p_ref, b_ref, o_ref):
    o_ref[...] = jnp.maximum(p_ref[0] + p_ref[1] + b_ref[...], 0.0)


def _tc_relu_combine(partials, b, block_rows=2000):
    d = partials.shape[2]
    return pl.pallas_call(
        _relu_combine_kernel,
        grid=(N // block_rows,),
        in_specs=[
            pl.BlockSpec((2, block_rows, d), lambda i: (0, i, 0)),
            pl.BlockSpec((1, d), lambda i: (0, 0)),
        ],
        out_specs=pl.BlockSpec((block_rows, d), lambda i: (i, 0)),
        out_shape=jax.ShapeDtypeStruct((N, d), jnp.float32),
    )(partials, b.reshape(1, d))


def _final_mm_kernel(p_ref, w_ref, b_ref, o_ref):
    agg = p_ref[0] + p_ref[1]
    o_ref[...] = jnp.dot(agg, w_ref[...],
                         preferred_element_type=jnp.float32,
                         precision=lax.Precision.HIGHEST) + b_ref[...]


def _tc_final_matmul(partials, w, b, block_rows=2000):
    d_in = partials.shape[2]
    d_out = w.shape[1]
    return pl.pallas_call(
        _final_mm_kernel,
        grid=(N // block_rows,),
        in_specs=[
            pl.BlockSpec((2, block_rows, d_in), lambda i: (0, i, 0)),
            pl.BlockSpec((d_in, d_out), lambda i: (0, 0)),
            pl.BlockSpec((1, d_out), lambda i: (0, 0)),
        ],
        out_specs=pl.BlockSpec((block_rows, d_out), lambda i: (i, 0)),
        out_shape=jax.ShapeDtypeStruct((N, d_out), jnp.float32),
    )(partials, w, b.reshape(1, d_out))


def _make_propagate():
    mesh = plsc.VectorSubcoreMesh(core_axis_name="c", subcore_axis_name="s")
    nch = D // LANES

    @functools.partial(
        pl.kernel,
        out_type=jax.ShapeDtypeStruct((NC, NP, D), jnp.float32),
        mesh=mesh,
        scratch_types=[
            pltpu.VMEM((K,), jnp.int32),
            pltpu.VMEM((K,), jnp.int32),
            pltpu.VMEM((K, LANES), jnp.float32),
            pltpu.VMEM((K, D), jnp.float32),
            pltpu.VMEM_SHARED((NP, D), jnp.float32),
            pltpu.SemaphoreType.DMA,
        ],
    )
    def prop(feat_hbm, src_hbm, dst_hbm, attr_hbm, out_hbm,
             srcv, dstv, attrv, rows, acc, sem):
        c = lax.axis_index("c")
        s = lax.axis_index("s")
        w = s * NC + c

        zero = jnp.zeros((LANES,), jnp.float32)

        @pl.loop(0, K)
        def _(r):
            for ch in range(nch):
                rows[r, pl.ds(ch * LANES, LANES)] = zero

        base_row = s * RPT
        for i in range(RPT // K):
            pltpu.sync_copy(rows, acc.at[pl.ds(base_row + i * K, K)])
        plsc.subcore_barrier()

        ebase = w * EW

        @pl.loop(0, NB)
        def _(bi):
            off = ebase + bi * K
            pltpu.sync_copy(src_hbm.at[pl.ds(off, K)], srcv)
            pltpu.sync_copy(dst_hbm.at[pl.ds(off, K)], dstv)
            pltpu.sync_copy(attr_hbm.at[pl.ds(off, K)], attrv)
            pltpu.async_copy(feat_hbm.at[srcv], rows, sem).wait()

            @pl.loop(0, K)
            def _(e):
                a = attrv[e, pl.ds(0, LANES)]
                for ch in range(nch):
                    sl = (e, pl.ds(ch * LANES, LANES))
                    rows[sl] = rows[sl] * a

            pltpu.sync_copy(rows, acc.at[dstv], add=True)

        plsc.subcore_barrier()

        for i in range(RPT // K):
            pltpu.sync_copy(acc.at[pl.ds(base_row + i * K, K)],
                            out_hbm.at[c, pl.ds(base_row + i * K, K)])

    return prop


_propagate = _make_propagate()


def kernel(x, edge_index, edge_attr, W1, b1, W2, b2):
    src = edge_index[0].astype(jnp.int32)
    dst = edge_index[1].astype(jnp.int32)
    attr16 = jnp.broadcast_to(edge_attr[:, None], (E, LANES))

    support1 = _tc_matmul(x, W1)
    partials1 = _propagate(support1, src, dst, attr16)
    h = _tc_relu_combine(partials1, b1)
    partials2 = _propagate(h, src, dst, attr16)
    return _tc_final_matmul(partials2, W2, b2)


# speedup vs baseline: 3.9524x; 1.2883x over previous
import functools

import jax
import jax.numpy as jnp
from jax import lax
from jax.experimental import pallas as pl
from jax.experimental.pallas import tpu as pltpu
from jax.experimental.pallas import tpu_sc as plsc

N = 10000
E = 320000
IN_CH = 128
HID_CH = 128
OUT_CH = 64

NC = 2
NS = 16
NW = NC * NS
EW = E // NW
K = 80
NB = EW // K
NP = 10240
RPT = NP // NS
LANES = 16
D = HID_CH


def _mm_kernel(x_ref, w_ref, o_ref):
    o_ref[...] = jnp.dot(x_ref[...], w_ref[...],
                         preferred_element_type=jnp.float32,
                         precision=lax.Precision.HIGHEST)


def _tc_matmul(x, w, block_rows=2000):
    n, d_in = x.shape
    d_out = w.shape[1]
    return pl.pallas_call(
        _mm_kernel,
        grid=(n // block_rows,),
        in_specs=[
            pl.BlockSpec((block_rows, d_in), lambda i: (i, 0)),
            pl.BlockSpec((d_in, d_out), lambda i: (0, 0)),
        ],
        out_specs=pl.BlockSpec((block_rows, d_out), lambda i: (i, 0)),
        out_shape=jax.ShapeDtypeStruct((n, d_out), jnp.float32),
    )(x, w)


def _relu_combine_kernel(p_ref, b_ref, o_ref):
    o_ref[...] = jnp.maximum(p_ref[0] + p_ref[1] + b_ref[...], 0.0)


def _tc_relu_combine(partials, b, block_rows=2000):
    d = partials.shape[2]
    return pl.pallas_call(
        _relu_combine_kernel,
        grid=(N // block_rows,),
        in_specs=[
            pl.BlockSpec((2, block_rows, d), lambda i: (0, i, 0)),
            pl.BlockSpec((1, d), lambda i: (0, 0)),
        ],
        out_specs=pl.BlockSpec((block_rows, d), lambda i: (i, 0)),
        out_shape=jax.ShapeDtypeStruct((N, d), jnp.float32),
    )(partials, b.reshape(1, d))


def _final_mm_kernel(p_ref, w_ref, b_ref, o_ref):
    agg = p_ref[0] + p_ref[1]
    o_ref[...] = jnp.dot(agg, w_ref[...],
                         preferred_element_type=jnp.float32,
                         precision=lax.Precision.HIGHEST) + b_ref[...]


def _tc_final_matmul(partials, w, b, block_rows=2000):
    d_in = partials.shape[2]
    d_out = w.shape[1]
    return pl.pallas_call(
        _final_mm_kernel,
        grid=(N // block_rows,),
        in_specs=[
            pl.BlockSpec((2, block_rows, d_in), lambda i: (0, i, 0)),
            pl.BlockSpec((d_in, d_out), lambda i: (0, 0)),
            pl.BlockSpec((1, d_out), lambda i: (0, 0)),
        ],
        out_specs=pl.BlockSpec((block_rows, d_out), lambda i: (i, 0)),
        out_shape=jax.ShapeDtypeStruct((N, d_out), jnp.float32),
    )(partials, w, b.reshape(1, d_out))


def _make_propagate():
    mesh = plsc.VectorSubcoreMesh(core_axis_name="c", subcore_axis_name="s")
    nch = D // LANES

    @functools.partial(
        pl.kernel,
        out_type=jax.ShapeDtypeStruct((NC, NP, D), jnp.float32),
        mesh=mesh,
        scratch_types=[
            pltpu.VMEM((K,), jnp.int32),
            pltpu.VMEM((K,), jnp.int32),
            pltpu.VMEM((K,), jnp.int32),
            pltpu.VMEM((K,), jnp.int32),
            pltpu.VMEM((K, LANES), jnp.float32),
            pltpu.VMEM((K, LANES), jnp.float32),
            pltpu.VMEM((K, D), jnp.float32),
            pltpu.VMEM((K, D), jnp.float32),
            pltpu.VMEM_SHARED((NP, D), jnp.float32),
            pltpu.SemaphoreType.DMA,
            pltpu.SemaphoreType.DMA,
        ],
    )
    def prop(feat_hbm, src_hbm, dst_hbm, attr_hbm, out_hbm,
             src0, src1, dst0, dst1, attr0, attr1, rows0, rows1,
             acc, sem0, sem1):
        c = lax.axis_index("c")
        s = lax.axis_index("s")
        w = s * NC + c

        zero = jnp.zeros((LANES,), jnp.float32)

        @pl.loop(0, K)
        def _(r):
            for ch in range(nch):
                rows0[r, pl.ds(ch * LANES, LANES)] = zero

        base_row = s * RPT
        for i in range(RPT // K):
            pltpu.sync_copy(rows0, acc.at[pl.ds(base_row + i * K, K)])
        plsc.subcore_barrier()

        ebase = w * EW

        def issue(off, srcv, dstv, attrv, rows, sem):
            pltpu.sync_copy(src_hbm.at[pl.ds(off, K)], srcv)
            pltpu.sync_copy(dst_hbm.at[pl.ds(off, K)], dstv)
            pltpu.sync_copy(attr_hbm.at[pl.ds(off, K)], attrv)
            pltpu.async_copy(feat_hbm.at[srcv], rows, sem)

        def process(srcv, dstv, attrv, rows, sem):
            pltpu.make_async_copy(feat_hbm.at[srcv], rows, sem).wait()

            @pl.loop(0, K)
            def _(e):
                a = attrv[e, pl.ds(0, LANES)]
                for ch in range(nch):
                    sl = (e, pl.ds(ch * LANES, LANES))
                    rows[sl] = rows[sl] * a

            pltpu.sync_copy(rows, acc.at[dstv], add=True)

        issue(ebase, src0, dst0, attr0, rows0, sem0)

        @pl.loop(0, (NB - 1) // 2)
        def _(i):
            off = ebase + 2 * i * K
            issue(off + K, src1, dst1, attr1, rows1, sem1)
            process(src0, dst0, attr0, rows0, sem0)
            issue(off + 2 * K, src0, dst0, attr0, rows0, sem0)
            process(src1, dst1, attr1, rows1, sem1)

        process(src0, dst0, attr0, rows0, sem0)

        plsc.subcore_barrier()

        for i in range(RPT // K):
            pltpu.sync_copy(acc.at[pl.ds(base_row + i * K, K)],
                            out_hbm.at[c, pl.ds(base_row + i * K, K)])

    return prop


_propagate = _make_propagate()


def kernel(x, edge_index, edge_attr, W1, b1, W2, b2):
    src = edge_index[0].astype(jnp.int32)
    dst = edge_index[1].astype(jnp.int32)
    attr16 = jnp.broadcast_to(edge_attr[:, None], (E, LANES))

    support1 = _tc_matmul(x, W1)
    partials1 = _propagate(support1, src, dst, attr16)
    h = _tc_relu_combine(partials1, b1)
    partials2 = _propagate(h, src, dst, attr16)
    return _tc_final_matmul(partials2, W2, b2)


# speedup vs baseline: 8.5771x; 2.1701x over previous
import functools

import jax
import jax.numpy as jnp
from jax import lax
from jax.experimental import pallas as pl
from jax.experimental.pallas import tpu as pltpu
from jax.experimental.pallas import tpu_sc as plsc

N = 10000
E = 320000
IN_CH = 128
HID_CH = 128
OUT_CH = 64

NC = 2
NS = 16
NW = NC * NS
EW = E // NW
K = 80
NB = EW // K
NP = 10240
RPT = NP // NS
LANES = 16
D = HID_CH


def _mm_kernel(x_ref, w_ref, o_ref):
    o_ref[...] = jnp.dot(x_ref[...], w_ref[...],
                         preferred_element_type=jnp.float32,
                         precision=lax.Precision.HIGHEST)


def _tc_matmul(x, w, block_rows=2000):
    n, d_in = x.shape
    d_out = w.shape[1]
    return pl.pallas_call(
        _mm_kernel,
        grid=(n // block_rows,),
        in_specs=[
            pl.BlockSpec((block_rows, d_in), lambda i: (i, 0)),
            pl.BlockSpec((d_in, d_out), lambda i: (0, 0)),
        ],
        out_specs=pl.BlockSpec((block_rows, d_out), lambda i: (i, 0)),
        out_shape=jax.ShapeDtypeStruct((n, d_out), jnp.float32),
    )(x, w)


def _relu_combine_kernel(p_ref, b_ref, o_ref):
    o_ref[...] = jnp.maximum(p_ref[0] + p_ref[1] + b_ref[...], 0.0)


def _tc_relu_combine(partials, b, block_rows=2000):
    d = partials.shape[2]
    return pl.pallas_call(
        _relu_combine_kernel,
        grid=(N // block_rows,),
        in_specs=[
            pl.BlockSpec((2, block_rows, d), lambda i: (0, i, 0)),
            pl.BlockSpec((1, d), lambda i: (0, 0)),
        ],
        out_specs=pl.BlockSpec((block_rows, d), lambda i: (i, 0)),
        out_shape=jax.ShapeDtypeStruct((N, d), jnp.float32),
    )(partials, b.reshape(1, d))


def _final_mm_kernel(p_ref, w_ref, b_ref, o_ref):
    agg = p_ref[0] + p_ref[1]
    o_ref[...] = jnp.dot(agg, w_ref[...],
                         preferred_element_type=jnp.float32,
                         precision=lax.Precision.HIGHEST) + b_ref[...]


def _tc_final_matmul(partials, w, b, block_rows=2000):
    d_in = partials.shape[2]
    d_out = w.shape[1]
    return pl.pallas_call(
        _final_mm_kernel,
        grid=(N // block_rows,),
        in_specs=[
            pl.BlockSpec((2, block_rows, d_in), lambda i: (0, i, 0)),
            pl.BlockSpec((d_in, d_out), lambda i: (0, 0)),
            pl.BlockSpec((1, d_out), lambda i: (0, 0)),
        ],
        out_specs=pl.BlockSpec((block_rows, d_out), lambda i: (i, 0)),
        out_shape=jax.ShapeDtypeStruct((N, d_out), jnp.float32),
    )(partials, w, b.reshape(1, d_out))


def _make_propagate():
    mesh = plsc.VectorSubcoreMesh(core_axis_name="c", subcore_axis_name="s")
    nch = D // LANES

    @functools.partial(
        pl.kernel,
        out_type=jax.ShapeDtypeStruct((NC, NP, D), jnp.float32),
        mesh=mesh,
        scratch_types=[
            pltpu.VMEM((EW,), jnp.int32),
            pltpu.VMEM((K,), jnp.int32),
            pltpu.VMEM((K,), jnp.int32),
            pltpu.VMEM((K,), jnp.float32),
            pltpu.VMEM((K,), jnp.float32),
            pltpu.VMEM((K, D), jnp.float32),
            pltpu.VMEM((K, D), jnp.float32),
            pltpu.VMEM_SHARED((NP, D), jnp.float32),
            pltpu.SemaphoreType.DMA,
            pltpu.SemaphoreType.DMA,
        ],
    )
    def prop(feat_hbm, src_hbm, dst_hbm, attr_hbm, out_hbm,
             srcall, dst0, dst1, attr0, attr1, rows0, rows1,
             acc, sem0, sem1):
        c = lax.axis_index("c")
        s = lax.axis_index("s")
        w = s * NC + c
        ebase = w * EW

        pltpu.sync_copy(src_hbm.at[pl.ds(ebase, EW)], srcall)

        zero = jnp.zeros((LANES,), jnp.float32)

        @pl.loop(0, K)
        def _(r):
            for ch in range(nch):
                rows0[r, pl.ds(ch * LANES, LANES)] = zero

        base_row = s * RPT
        for i in range(RPT // K):
            pltpu.sync_copy(rows0, acc.at[pl.ds(base_row + i * K, K)])
        plsc.subcore_barrier()

        def issue(bi, dstv, attrv, rows, sem):
            off = ebase + bi * K
            loff = bi * K
            pltpu.async_copy(dst_hbm.at[pl.ds(off, K)], dstv, sem)
            pltpu.async_copy(attr_hbm.at[pl.ds(off, K)], attrv, sem)
            pltpu.async_copy(feat_hbm.at[srcall.at[pl.ds(loff, K)]], rows, sem)

        def process(bi, dstv, attrv, rows, sem):
            off = ebase + bi * K
            loff = bi * K
            pltpu.make_async_copy(dst_hbm.at[pl.ds(off, K)], dstv, sem).wait()
            pltpu.make_async_copy(attr_hbm.at[pl.ds(off, K)], attrv, sem).wait()
            pltpu.make_async_copy(
                feat_hbm.at[srcall.at[pl.ds(loff, K)]], rows, sem).wait()

            @pl.loop(0, K)
            def _(e):
                a = attrv[pl.ds(e, 1)][0]
                for ch in range(nch):
                    sl = (e, pl.ds(ch * LANES, LANES))
                    rows[sl] = rows[sl] * a

            pltpu.sync_copy(rows, acc.at[dstv], add=True)

        issue(0, dst0, attr0, rows0, sem0)

        @pl.loop(0, (NB - 1) // 2)
        def _(i):
            bi = 2 * i
            issue(bi + 1, dst1, attr1, rows1, sem1)
            process(bi, dst0, attr0, rows0, sem0)
            issue(bi + 2, dst0, attr0, rows0, sem0)
            process(bi + 1, dst1, attr1, rows1, sem1)

        process(NB - 1, dst0, attr0, rows0, sem0)

        plsc.subcore_barrier()

        for i in range(RPT // K):
            pltpu.sync_copy(acc.at[pl.ds(base_row + i * K, K)],
                            out_hbm.at[c, pl.ds(base_row + i * K, K)])

    return prop


_propagate = _make_propagate()


def kernel(x, edge_index, edge_attr, W1, b1, W2, b2):
    src = edge_index[0].astype(jnp.int32)
    dst = edge_index[1].astype(jnp.int32)

    support1 = _tc_matmul(x, W1)
    partials1 = _propagate(support1, src, dst, edge_attr)
    h = _tc_relu_combine(partials1, b1)
    partials2 = _propagate(h, src, dst, edge_attr)
    return _tc_final_matmul(partials2, W2, b2)


# speedup vs baseline: 10.4651x; 1.2201x over previous
import functools

import jax
import jax.numpy as jnp
from jax import lax
from jax.experimental import pallas as pl
from jax.experimental.pallas import tpu as pltpu
from jax.experimental.pallas import tpu_sc as plsc

N = 10000
E = 320000
IN_CH = 128
HID_CH = 128
OUT_CH = 64

NC = 2
NS = 16
NW = NC * NS
EW = E // NW
K = 80
NB = EW // K
NP = 10240
RPT = NP // NS
LANES = 16
D = HID_CH


def _mm_kernel(x_ref, w_ref, o_ref):
    o_ref[...] = jnp.dot(x_ref[...], w_ref[...],
                         preferred_element_type=jnp.float32,
                         precision=lax.Precision.HIGHEST)


def _tc_matmul(x, w, block_rows=2000):
    n, d_in = x.shape
    d_out = w.shape[1]
    return pl.pallas_call(
        _mm_kernel,
        grid=(n // block_rows,),
        in_specs=[
            pl.BlockSpec((block_rows, d_in), lambda i: (i, 0)),
            pl.BlockSpec((d_in, d_out), lambda i: (0, 0)),
        ],
        out_specs=pl.BlockSpec((block_rows, d_out), lambda i: (i, 0)),
        out_shape=jax.ShapeDtypeStruct((n, d_out), jnp.float32),
    )(x, w)


def _relu_combine_kernel(p_ref, b_ref, o_ref):
    o_ref[...] = jnp.maximum(p_ref[0] + p_ref[1] + b_ref[...], 0.0)


def _tc_relu_combine(partials, b, block_rows=2000):
    d = partials.shape[2]
    return pl.pallas_call(
        _relu_combine_kernel,
        grid=(N // block_rows,),
        in_specs=[
            pl.BlockSpec((2, block_rows, d), lambda i: (0, i, 0)),
            pl.BlockSpec((1, d), lambda i: (0, 0)),
        ],
        out_specs=pl.BlockSpec((block_rows, d), lambda i: (i, 0)),
        out_shape=jax.ShapeDtypeStruct((N, d), jnp.float32),
    )(partials, b.reshape(1, d))


def _final_mm_kernel(p_ref, w_ref, b_ref, o_ref):
    agg = p_ref[0] + p_ref[1]
    o_ref[...] = jnp.dot(agg, w_ref[...],
                         preferred_element_type=jnp.float32,
                         precision=lax.Precision.HIGHEST) + b_ref[...]


def _tc_final_matmul(partials, w, b, block_rows=2000):
    d_in = partials.shape[2]
    d_out = w.shape[1]
    return pl.pallas_call(
        _final_mm_kernel,
        grid=(N // block_rows,),
        in_specs=[
            pl.BlockSpec((2, block_rows, d_in), lambda i: (0, i, 0)),
            pl.BlockSpec((d_in, d_out), lambda i: (0, 0)),
            pl.BlockSpec((1, d_out), lambda i: (0, 0)),
        ],
        out_specs=pl.BlockSpec((block_rows, d_out), lambda i: (i, 0)),
        out_shape=jax.ShapeDtypeStruct((N, d_out), jnp.float32),
    )(partials, w, b.reshape(1, d_out))


def _make_propagate():
    mesh = plsc.VectorSubcoreMesh(core_axis_name="c", subcore_axis_name="s")
    nch = D // LANES

    @functools.partial(
        pl.kernel,
        out_type=jax.ShapeDtypeStruct((NC, NP, D), jnp.float32),
        mesh=mesh,
        scratch_types=[
            pltpu.VMEM((EW,), jnp.int32),
            pltpu.VMEM((K,), jnp.int32),
            pltpu.VMEM((K,), jnp.int32),
            pltpu.VMEM((K,), jnp.float32),
            pltpu.VMEM((K,), jnp.float32),
            pltpu.VMEM((K, D), jnp.float32),
            pltpu.VMEM((K, D), jnp.float32),
            pltpu.VMEM_SHARED((NP, D), jnp.float32),
            pltpu.SemaphoreType.DMA,
            pltpu.SemaphoreType.DMA,
        ],
    )
    def prop(feat_hbm, src_hbm, dst_hbm, attr_hbm, out_hbm,
             srcall, dst0, dst1, attr0, attr1, rows0, rows1,
             acc, sem0, sem1):
        c = lax.axis_index("c")
        s = lax.axis_index("s")
        w = s * NC + c
        ebase = w * EW

        pltpu.sync_copy(src_hbm.at[pl.ds(ebase, EW)], srcall)

        zero = jnp.zeros((LANES,), jnp.float32)

        @plsc.parallel_loop(0, K, unroll=4)
        def _(r):
            for ch in range(nch):
                rows0[r, pl.ds(ch * LANES, LANES)] = zero

        base_row = s * RPT
        for i in range(RPT // K):
            pltpu.sync_copy(rows0, acc.at[pl.ds(base_row + i * K, K)])
        plsc.subcore_barrier()

        def issue(bi, dstv, attrv, rows, sem):
            off = ebase + bi * K
            loff = bi * K
            pltpu.async_copy(dst_hbm.at[pl.ds(off, K)], dstv, sem)
            pltpu.async_copy(attr_hbm.at[pl.ds(off, K)], attrv, sem)
            pltpu.async_copy(feat_hbm.at[srcall.at[pl.ds(loff, K)]], rows, sem)

        def process(bi, dstv, attrv, rows, sem):
            off = ebase + bi * K
            loff = bi * K
            pltpu.make_async_copy(dst_hbm.at[pl.ds(off, K)], dstv, sem).wait()
            pltpu.make_async_copy(attr_hbm.at[pl.ds(off, K)], attrv, sem).wait()
            pltpu.make_async_copy(
                feat_hbm.at[srcall.at[pl.ds(loff, K)]], rows, sem).wait()

            @plsc.parallel_loop(0, K, unroll=4)
            def _(e):
                a = attrv[pl.ds(e, 1)][0]
                for ch in range(nch):
                    sl = (e, pl.ds(ch * LANES, LANES))
                    rows[sl] = rows[sl] * a

            pltpu.sync_copy(rows, acc.at[dstv], add=True)

        issue(0, dst0, attr0, rows0, sem0)

        @pl.loop(0, (NB - 1) // 2)
        def _(i):
            bi = 2 * i
            issue(bi + 1, dst1, attr1, rows1, sem1)
            process(bi, dst0, attr0, rows0, sem0)
            issue(bi + 2, dst0, attr0, rows0, sem0)
            process(bi + 1, dst1, attr1, rows1, sem1)

        process(NB - 1, dst0, attr0, rows0, sem0)

        plsc.subcore_barrier()

        for i in range(RPT // K):
            pltpu.sync_copy(acc.at[pl.ds(base_row + i * K, K)],
                            out_hbm.at[c, pl.ds(base_row + i * K, K)])

    return prop


_propagate = _make_propagate()


def kernel(x, edge_index, edge_attr, W1, b1, W2, b2):
    src = edge_index[0].astype(jnp.int32)
    dst = edge_index[1].astype(jnp.int32)

    support1 = _tc_matmul(x, W1)
    partials1 = _propagate(support1, src, dst, edge_attr)
    h = _tc_relu_combine(partials1, b1)
    partials2 = _propagate(h, src, dst, edge_attr)
    return _tc_final_matmul(partials2, W2, b2)


# speedup vs baseline: 11.6868x; 1.1167x over previous
import functools

import jax
import jax.numpy as jnp
from jax import lax
from jax.experimental import pallas as pl
from jax.experimental.pallas import tpu as pltpu
from jax.experimental.pallas import tpu_sc as plsc

N = 10000
E = 320000
IN_CH = 128
HID_CH = 128
OUT_CH = 64

NC = 2
NS = 16
NW = NC * NS
EW = E // NW
K = 80
NB = EW // K
NP = 10240
RPT = NP // NS
LANES = 16
D = HID_CH


def _mm_kernel(x_ref, w_ref, o_ref):
    o_ref[...] = jnp.dot(x_ref[...], w_ref[...],
                         preferred_element_type=jnp.float32,
                         precision=lax.Precision.HIGHEST)


def _tc_matmul(x, w, block_rows=2000):
    n, d_in = x.shape
    d_out = w.shape[1]
    return pl.pallas_call(
        _mm_kernel,
        grid=(n // block_rows,),
        in_specs=[
            pl.BlockSpec((block_rows, d_in), lambda i: (i, 0)),
            pl.BlockSpec((d_in, d_out), lambda i: (0, 0)),
        ],
        out_specs=pl.BlockSpec((block_rows, d_out), lambda i: (i, 0)),
        out_shape=jax.ShapeDtypeStruct((n, d_out), jnp.float32),
    )(x, w)


def _relu_combine_kernel(p_ref, b_ref, o_ref):
    o_ref[...] = jnp.maximum(p_ref[0] + p_ref[1] + b_ref[...], 0.0)


def _tc_relu_combine(partials, b, block_rows=2000):
    d = partials.shape[2]
    return pl.pallas_call(
        _relu_combine_kernel,
        grid=(N // block_rows,),
        in_specs=[
            pl.BlockSpec((2, block_rows, d), lambda i: (0, i, 0)),
            pl.BlockSpec((1, d), lambda i: (0, 0)),
        ],
        out_specs=pl.BlockSpec((block_rows, d), lambda i: (i, 0)),
        out_shape=jax.ShapeDtypeStruct((N, d), jnp.float32),
    )(partials, b.reshape(1, d))


def _final_mm_kernel(p_ref, w_ref, b_ref, o_ref):
    agg = p_ref[0] + p_ref[1]
    o_ref[...] = jnp.dot(agg, w_ref[...],
                         preferred_element_type=jnp.float32,
                         precision=lax.Precision.HIGHEST) + b_ref[...]


def _tc_final_matmul(partials, w, b, block_rows=2000):
    d_in = partials.shape[2]
    d_out = w.shape[1]
    return pl.pallas_call(
        _final_mm_kernel,
        grid=(N // block_rows,),
        in_specs=[
            pl.BlockSpec((2, block_rows, d_in), lambda i: (0, i, 0)),
            pl.BlockSpec((d_in, d_out), lambda i: (0, 0)),
            pl.BlockSpec((1, d_out), lambda i: (0, 0)),
        ],
        out_specs=pl.BlockSpec((block_rows, d_out), lambda i: (i, 0)),
        out_shape=jax.ShapeDtypeStruct((N, d_out), jnp.float32),
    )(partials, w, b.reshape(1, d_out))


def _make_propagate():
    mesh = plsc.VectorSubcoreMesh(core_axis_name="c", subcore_axis_name="s")
    nch = D // LANES

    @functools.partial(
        pl.kernel,
        out_type=jax.ShapeDtypeStruct((NC, NP, D), jnp.float32),
        mesh=mesh,
        scratch_types=[
            pltpu.VMEM((EW,), jnp.int32),
            pltpu.VMEM((K, D), jnp.float32),
            pltpu.VMEM((K, D), jnp.float32),
            pltpu.VMEM((K, D), jnp.float32),
            pltpu.VMEM((K,), jnp.int32),
            pltpu.VMEM((K,), jnp.int32),
            pltpu.VMEM((K,), jnp.int32),
            pltpu.VMEM((K,), jnp.float32),
            pltpu.VMEM((K,), jnp.float32),
            pltpu.VMEM((K,), jnp.float32),
            pltpu.VMEM_SHARED((NP, D), jnp.float32),
            pltpu.SemaphoreType.DMA,
            pltpu.SemaphoreType.DMA,
            pltpu.SemaphoreType.DMA,
            pltpu.SemaphoreType.DMA,
            pltpu.SemaphoreType.DMA,
            pltpu.SemaphoreType.DMA,
            pltpu.SemaphoreType.DMA,
            pltpu.SemaphoreType.DMA,
            pltpu.SemaphoreType.DMA,
        ],
    )
    def prop(feat_hbm, src_hbm, dst_hbm, attr_hbm, out_hbm,
             srcall, rows0, rows1, rows2, dstb0, dstb1, dstb2,
             attrb0, attrb1, attrb2,
             acc, g0, g1, g2, e0, e1, e2, sc0, sc1, sc2):
        c = lax.axis_index("c")
        s = lax.axis_index("s")
        w = s * NC + c
        ebase = w * EW

        pltpu.sync_copy(src_hbm.at[pl.ds(ebase, EW)], srcall)

        rows = (rows0, rows1, rows2)
        dstb = (dstb0, dstb1, dstb2)
        attrb = (attrb0, attrb1, attrb2)
        gsem = (g0, g1, g2)
        esem = (e0, e1, e2)
        ssem = (sc0, sc1, sc2)

        zero = jnp.zeros((LANES,), jnp.float32)

        @plsc.parallel_loop(0, K, unroll=4)
        def _(r):
            for ch in range(nch):
                rows0[r, pl.ds(ch * LANES, LANES)] = zero

        base_row = s * RPT
        for i in range(RPT // K):
            pltpu.sync_copy(rows0, acc.at[pl.ds(base_row + i * K, K)])
        plsc.subcore_barrier()

        def issue(b, sl):
            pltpu.async_copy(
                feat_hbm.at[srcall.at[pl.ds(b * K, K)]], rows[sl], gsem[sl])
            pltpu.async_copy(
                dst_hbm.at[pl.ds(ebase + b * K, K)], dstb[sl], esem[sl])
            pltpu.async_copy(
                attr_hbm.at[pl.ds(ebase + b * K, K)], attrb[sl], esem[sl])

        def wait_gather(b, sl):
            pltpu.make_async_copy(
                feat_hbm.at[srcall.at[pl.ds(b * K, K)]],
                rows[sl], gsem[sl]).wait()
            pltpu.make_async_copy(
                dst_hbm.at[pl.ds(ebase + b * K, K)], dstb[sl], esem[sl]).wait()
            pltpu.make_async_copy(
                attr_hbm.at[pl.ds(ebase + b * K, K)],
                attrb[sl], esem[sl]).wait()

        def scale(b, sl):
            r = rows[sl]
            a_ref = attrb[sl]

            @plsc.parallel_loop(0, K, unroll=4)
            def _(e):
                a = a_ref[pl.ds(e, 1)][0]
                for ch in range(nch):
                    idx = (e, pl.ds(ch * LANES, LANES))
                    r[idx] = r[idx] * a

        def scatter(b, sl):
            pltpu.async_copy(
                rows[sl], acc.at[dstb[sl].at[pl.ds(0, K)]], ssem[sl],
                add=True)

        def wait_scatter(b, sl):
            pltpu.make_async_copy(
                rows[sl], acc.at[dstb[sl].at[pl.ds(0, K)]],
                ssem[sl]).wait()

        issue(0, 0)
        issue(1, 1)

        wait_gather(0, 0); scale(0, 0); scatter(0, 0)
        issue(2, 2)
        wait_gather(1, 1); scale(1, 1); scatter(1, 1)
        wait_scatter(0, 0); issue(3, 0)
        wait_gather(2, 2); scale(2, 2); scatter(2, 2)
        wait_scatter(1, 1); issue(4, 1)

        @pl.loop(0, (NB - 5) // 3)
        def _(i):
            b = 3 * i + 3
            wait_gather(b, 0); scale(b, 0); scatter(b, 0)
            wait_scatter(b - 1, 2); issue(b + 2, 2)
            wait_gather(b + 1, 1); scale(b + 1, 1); scatter(b + 1, 1)
            wait_scatter(b, 0); issue(b + 3, 0)
            wait_gather(b + 2, 2); scale(b + 2, 2); scatter(b + 2, 2)
            wait_scatter(b + 1, 1); issue(b + 4, 1)

        wait_gather(NB - 2, 0); scale(NB - 2, 0); scatter(NB - 2, 0)
        wait_scatter(NB - 3, 2)
        wait_gather(NB - 1, 1); scale(NB - 1, 1); scatter(NB - 1, 1)
        wait_scatter(NB - 2, 0)
        wait_scatter(NB - 1, 1)

        plsc.subcore_barrier()

        for i in range(RPT // K):
            pltpu.sync_copy(acc.at[pl.ds(base_row + i * K, K)],
                            out_hbm.at[c, pl.ds(base_row + i * K, K)])

    return prop


_propagate = _make_propagate()


def kernel(x, edge_index, edge_attr, W1, b1, W2, b2):
    src = edge_index[0].astype(jnp.int32)
    dst = edge_index[1].astype(jnp.int32)

    support1 = _tc_matmul(x, W1)
    partials1 = _propagate(support1, src, dst, edge_attr)
    h = _tc_relu_combine(partials1, b1)
    partials2 = _propagate(h, src, dst, edge_attr)
    return _tc_final_matmul(partials2, W2, b2)
